# baseline (speedup 1.0000x reference)
import jax, jax.numpy as jnp
from jax.experimental import pallas as pl

N = 10000


def _gcn_conv(x, W, b, src, dst, n):
    xw = x @ W
    loop = jnp.arange(n, dtype=src.dtype)
    src2 = jnp.concatenate([src, loop])
    dst2 = jnp.concatenate([dst, loop])
    ones = jnp.ones(src2.shape[0], dtype=x.dtype)
    deg = jax.ops.segment_sum(ones, dst2, num_segments=n)
    dinv = jnp.where(deg > 0, 1.0 / jnp.sqrt(deg), 0.0)
    norm = dinv[src2] * dinv[dst2]
    msg = norm[:, None] * jnp.take(xw, src2, axis=0)
    out = jax.ops.segment_sum(msg, dst2, num_segments=n)
    return out + b


def _batchnorm(x, g, beta, eps=1e-5):
    mu = x.mean(axis=0)
    var = x.var(axis=0)
    return g * (x - mu) / jnp.sqrt(var + eps) + beta


def kernel(x, edge_index, W1, b1, g1, be1, W2, b2, g2, be2, W3, b3, g3, be3, Wc1, bc1, Wc2, bc2, Wr1, br1, Wr2, br2):
    src = edge_index[0]
    dst = edge_index[1]
    h = jax.nn.relu(_batchnorm(_gcn_conv(x, W1, b1, src, dst, N), g1, be1))
    h = jax.nn.relu(_batchnorm(_gcn_conv(h, W2, b2, src, dst, N), g2, be2))
    h = _batchnorm(_gcn_conv(h, W3, b3, src, dst, N), g3, be3)
    h = h.mean(axis=0, keepdims=True)
    class_logits = jax.nn.relu(h @ Wc1 + bc1) @ Wc2 + bc2
    rul_pred = jax.nn.relu(h @ Wr1 + br1) @ Wr2 + br2
    return (class_logits, rul_pred)
